# Initial kernel scaffold; baseline (speedup 1.0000x reference)
import jax
import jax.numpy as jnp
from jax.experimental import pallas as pl


def kernel(anchors_left, anchors_right, objectness_left, objectness_right, box_regression_left, box_regression_right):
    raise NotImplementedError("write your pallas kernel here")


# speedup vs baseline: 9.7660x; 9.7660x over previous
import functools

import jax
import jax.numpy as jnp
import numpy as np
from jax import lax
from jax.experimental import pallas as pl
from jax.experimental.pallas import tpu as pltpu
from jax.experimental.pallas import tpu_sc as plsc

NUM = 49152
NB = NUM // 16
PRE = 2000
NV = PRE // 16
POST = 1000
OUTP = 1008
NOV = OUTP // 16
TAU = 0.7
CLIP = float(np.log(1000.0 / 16.0))
IMG_MAX = 1023.0
NEG = -1e9


def _kernel_body(obj_l, obj_r, anch_l, anch_r, reg_l, reg_r,
                 obl, osl, obr, osr,
                 buf, skeyA, skeyB, sidxA, sidxB, gidx,
                 hist, totals, base,
                 tdx, tdy, tdw, tdh, gidx2, px1, py1, px2, py2, areas, msc,
                 sup, keep, ob, os, shkeep):
    img = lax.axis_index("c")
    tid = lax.axis_index("s")
    iota = lax.iota(jnp.int32, 16)
    ones_i = jnp.full((16,), 1, jnp.int32)

    def clear_hist():
        def row(r, _):
            hist[pl.ds(r * 16, 16)] = jnp.zeros((16,), jnp.int32)
            return 0
        lax.fori_loop(0, 256, row, 0)

    def reduce_hist():
        def col(c, _):
            def row(r, acc):
                return acc + hist[pl.ds(r * 256 + c * 16, 16)]
            acc = lax.fori_loop(0, 16, row, jnp.zeros((16,), jnp.int32))
            totals[pl.ds(c * 16, 16)] = acc
            return 0
        lax.fori_loop(0, 16, col, 0)

    def ranks(d):
        fwd = jnp.zeros((16,), jnp.int32)
        bwd = jnp.zeros((16,), jnp.int32)
        for s in range(1, 16):
            rolled = d.at[(iota - s) & 15].get(mode="promise_in_bounds")
            eq = (d == rolled).astype(jnp.int32)
            fwd = fwd + jnp.where(iota >= s, eq, 0)
            bwd = bwd + jnp.where(iota < s, eq, 0)
        return fwd, bwd

    def select_topk(obj_ref):
        base_off = img * NUM
        pltpu.sync_copy(obj_ref.at[pl.ds(base_off, NUM)], buf)

        def sig(v, _):
            x = buf[pl.ds(v * 16, 16)]
            buf[pl.ds(v * 16, 16)] = 1.0 / (1.0 + jnp.exp(-x))
            return 0
        lax.fori_loop(0, NB, sig, 0)

        def level(lvl, carry):
            prefix, need = carry
            shift = 24 - lvl * 8
            clear_hist()

            def hpass(v, _):
                k = lax.bitcast_convert_type(buf[pl.ds(v * 16, 16)], jnp.int32)
                hk = k >> shift
                m = (hk >> 8) == prefix
                d = hk & 255
                plsc.addupdate_scatter(hist, [iota * 256 + d], ones_i, mask=m)
                return 0
            lax.fori_loop(0, NB, hpass, 0)
            reduce_hist()

            def chunk(ci, c):
                found, dstar, above, cum = c
                dbase = 240 - ci * 16
                t = totals[pl.ds(dbase, 16)]
                tr = lax.rev(t, (0,))
                cs = plsc.cumsum(tr) + cum
                hit = cs >= need
                anyhit = jnp.max(hit.astype(jnp.int32))
                f = jnp.max(plsc.all_reduce_ffs(hit))
                lane_eq = iota == f
                cs_at = jnp.max(jnp.where(lane_eq, cs, 0))
                t_at = jnp.max(jnp.where(lane_eq, tr, 0))
                d_cand = dbase + 15 - f
                a_cand = cs_at - t_at
                take = (anyhit > 0) & (found == 0)
                return (found | anyhit,
                        jnp.where(take, d_cand, dstar),
                        jnp.where(take, a_cand, above),
                        cum + jnp.sum(t))
            found, dstar, above, _ = lax.fori_loop(
                0, 16, chunk,
                (jnp.int32(0), jnp.int32(0), jnp.int32(0), jnp.int32(0)))
            return (prefix * 256 + dstar, need - above)

        T, tie_take = lax.fori_loop(0, 4, level, (jnp.int32(0), jnp.int32(PRE)))
        G = PRE - tie_take

        def cpass(v, carry):
            pgt, pt = carry
            s = buf[pl.ds(v * 16, 16)]
            k = lax.bitcast_convert_type(s, jnp.int32)
            gt = k > T
            tie = k == T
            gti = gt.astype(jnp.int32)
            tii = tie.astype(jnp.int32)
            cg = plsc.cumsum(gti)
            ct = plsc.cumsum(tii)
            idxv = v * 16 + iota
            posg = pgt + cg - 1
            post_ = G + pt + ct - 1
            plsc.store_scatter(skeyA, [posg], s, mask=gt)
            plsc.store_scatter(sidxA, [posg], idxv, mask=gt)
            mt = tie & (post_ < PRE)
            plsc.store_scatter(skeyA, [post_], s, mask=mt)
            plsc.store_scatter(sidxA, [post_], idxv, mask=mt)
            return (pgt + jnp.sum(gti), pt + jnp.sum(tii))
        lax.fori_loop(0, NB, cpass, (jnp.int32(0), jnp.int32(0)))

    def sort_pass(src_k, src_i, dst_k, dst_i, shift):
        clear_hist()

        def hpass(v, _):
            k = lax.bitcast_convert_type(src_k[pl.ds(v * 16, 16)], jnp.int32)
            dp = 255 - ((k >> shift) & 255)
            plsc.addupdate_scatter(hist, [iota * 256 + dp], ones_i)
            return 0
        lax.fori_loop(0, NV, hpass, 0)
        reduce_hist()

        def escan(c, carry):
            t = totals[pl.ds(c * 16, 16)]
            inc = plsc.cumsum(t)
            base[pl.ds(c * 16, 16)] = inc - t + carry
            return carry + jnp.sum(t)
        lax.fori_loop(0, 16, escan, jnp.int32(0))

        def spass(v, _):
            kf = src_k[pl.ds(v * 16, 16)]
            k = lax.bitcast_convert_type(kf, jnp.int32)
            dp = 255 - ((k >> shift) & 255)
            fwd, bwd = ranks(dp)
            bv = plsc.load_gather(base, [dp])
            pos = bv + fwd
            plsc.store_scatter(dst_k, [pos], kf)
            plsc.store_scatter(dst_i, [pos], src_i[pl.ds(v * 16, 16)])
            plsc.addupdate_scatter(base, [dp], fwd + 1, mask=(bwd == 0))
            return 0
        lax.fori_loop(0, NV, spass, 0)

    def gather_col(tbl_ref, col, dst):
        def gpre(v, _):
            gidx2[pl.ds(v * 16, 16)] = gidx[pl.ds(v * 16, 16)] * 4 + col
            return 0
        lax.fori_loop(0, NV, gpre, 0)

        def chunkg(ci, _):
            pltpu.sync_copy(tbl_ref.at[gidx2.at[pl.ds(ci * 128, 128)]],
                            dst.at[pl.ds(ci * 128, 128)])
            return 0
        lax.fori_loop(0, 15, chunkg, 0)
        pltpu.sync_copy(tbl_ref.at[gidx2.at[pl.ds(1920, 80)]],
                        dst.at[pl.ds(1920, 80)])

    def decode(anch_ref, reg_ref):
        def gpre(v, _):
            gidx[pl.ds(v * 16, 16)] = sidxA[pl.ds(v * 16, 16)] + img * NUM
            return 0
        lax.fori_loop(0, NV, gpre, 0)
        gather_col(anch_ref, 0, px1)
        gather_col(anch_ref, 1, py1)
        gather_col(anch_ref, 2, px2)
        gather_col(anch_ref, 3, py2)
        gather_col(reg_ref, 0, tdx)
        gather_col(reg_ref, 1, tdy)
        gather_col(reg_ref, 2, tdw)
        gather_col(reg_ref, 3, tdh)

        def dec(v, _):
            sl16 = pl.ds(v * 16, 16)
            ax1 = px1[sl16]
            ay1 = py1[sl16]
            ax2 = px2[sl16]
            ay2 = py2[sl16]
            dx = tdx[sl16]
            dy = tdy[sl16]
            dw = tdw[sl16]
            dh = tdh[sl16]
            w = ax2 - ax1 + 1.0
            h = ay2 - ay1 + 1.0
            cx = ax1 + 0.5 * w
            cy = ay1 + 0.5 * h
            dwc = jnp.minimum(dw, CLIP)
            dhc = jnp.minimum(dh, CLIP)
            pcx = dx * w + cx
            pcy = dy * h + cy
            pw = jnp.exp(dwc) * w
            ph = jnp.exp(dhc) * h
            x1 = pcx - 0.5 * pw
            y1 = pcy - 0.5 * ph
            x2 = pcx + 0.5 * pw - 1.0
            y2 = pcy + 0.5 * ph - 1.0
            x1 = jnp.minimum(jnp.maximum(x1, 0.0), IMG_MAX)
            y1 = jnp.minimum(jnp.maximum(y1, 0.0), IMG_MAX)
            x2 = jnp.minimum(jnp.maximum(x2, 0.0), IMG_MAX)
            y2 = jnp.minimum(jnp.maximum(y2, 0.0), IMG_MAX)
            ws = x2 - x1 + 1.0
            hs = y2 - y1 + 1.0
            keepm = (ws >= 0.0) & (hs >= 0.0)
            sv = skeyA[sl16]
            px1[sl16] = x1
            py1[sl16] = y1
            px2[sl16] = x2
            py2[sl16] = y2
            areas[sl16] = ws * hs
            msc[sl16] = jnp.where(keepm, sv, NEG)
            return 0
        lax.fori_loop(0, NV, dec, 0)

    def run_side(obj_ref, anch_ref, reg_ref):
        select_topk(obj_ref)
        sort_pass(skeyA, sidxA, skeyB, sidxB, 0)
        sort_pass(skeyB, sidxB, skeyA, sidxA, 8)
        sort_pass(skeyA, sidxA, skeyB, sidxB, 16)
        sort_pass(skeyB, sidxB, skeyA, sidxA, 24)
        decode(anch_ref, reg_ref)
        def kinit(i, _):
            keep[pl.ds(i * 16, 16)] = jnp.zeros((16,), jnp.int32)
            return 0
        lax.fori_loop(0, NOV, kinit, 0)

    def nms():
        def sinit(v, _):
            sup[pl.ds(v * 16, 16)] = jnp.zeros((16,), jnp.int32)
            return 0
        lax.fori_loop(0, NV, sinit, 0)

        def outer(v, kcount):
            def inner(l, kc):
                supv = sup[pl.ds(v * 16, 16)]
                lm = iota == l
                sj = jnp.max(jnp.where(lm, supv, 0))
                kept = sj == 0

                @pl.when(kept)
                def _():
                    fneg = jnp.full((16,), -3.4e38, jnp.float32)
                    bx1 = jnp.max(jnp.where(lm, px1[pl.ds(v * 16, 16)], fneg))
                    by1 = jnp.max(jnp.where(lm, py1[pl.ds(v * 16, 16)], fneg))
                    bx2 = jnp.max(jnp.where(lm, px2[pl.ds(v * 16, 16)], fneg))
                    by2 = jnp.max(jnp.where(lm, py2[pl.ds(v * 16, 16)], fneg))
                    ba = jnp.max(jnp.where(lm, areas[pl.ds(v * 16, 16)], fneg))
                    j = v * 16 + l
                    wm = (iota == 0) & (kc < OUTP)
                    plsc.store_scatter(keep, [jnp.full((16,), kc, jnp.int32)],
                                       jnp.full((16,), j, jnp.int32), mask=wm)

                    def supdate(u, _):
                        ux1 = px1[pl.ds(u * 16, 16)]
                        uy1 = py1[pl.ds(u * 16, 16)]
                        ux2 = px2[pl.ds(u * 16, 16)]
                        uy2 = py2[pl.ds(u * 16, 16)]
                        xx1 = jnp.maximum(bx1, ux1)
                        yy1 = jnp.maximum(by1, uy1)
                        xx2 = jnp.minimum(bx2, ux2)
                        yy2 = jnp.minimum(by2, uy2)
                        iw = jnp.maximum(xx2 - xx1 + 1.0, 0.0)
                        ih = jnp.maximum(yy2 - yy1 + 1.0, 0.0)
                        inter = iw * ih
                        iou = inter / (ba + areas[pl.ds(u * 16, 16)] - inter)
                        s_ = (iou > TAU).astype(jnp.int32)
                        sup[pl.ds(u * 16, 16)] = sup[pl.ds(u * 16, 16)] | s_
                        return 0
                    lax.fori_loop(0, NV, supdate, 0)

                return kc + jnp.where(kept, 1, 0)
            return lax.fori_loop(0, 16, inner, kcount)
        lax.fori_loop(0, NV, outer, jnp.int32(0))

    def gather_out(outb_ref, outs_ref):
        def g(i, _):
            idxv = keep[pl.ds(i * 16, 16)]
            ob[pl.ds(0 * OUTP + i * 16, 16)] = plsc.load_gather(px1, [idxv])
            ob[pl.ds(1 * OUTP + i * 16, 16)] = plsc.load_gather(py1, [idxv])
            ob[pl.ds(2 * OUTP + i * 16, 16)] = plsc.load_gather(px2, [idxv])
            ob[pl.ds(3 * OUTP + i * 16, 16)] = plsc.load_gather(py2, [idxv])
            os[pl.ds(i * 16, 16)] = plsc.load_gather(msc, [idxv])
            return 0
        lax.fori_loop(0, NOV, g, 0)
        pltpu.sync_copy(ob, outb_ref.at[img])
        pltpu.sync_copy(os, outs_ref.at[img])

    @pl.when(tid == 0)
    def _():
        run_side(obj_l, anch_l, reg_l)
        nms()
        pltpu.sync_copy(keep, shkeep)

    @pl.when(tid == 8)
    def _():
        run_side(obj_r, anch_r, reg_r)

    plsc.subcore_barrier()

    @pl.when(tid == 0)
    def _():
        gather_out(obl, osl)

    @pl.when(tid == 8)
    def _():
        pltpu.sync_copy(shkeep, keep)
        gather_out(obr, osr)


@jax.jit
def _run(obj_l, obj_r, anch_l, anch_r, reg_l, reg_r):
    scmesh = plsc.VectorSubcoreMesh(core_axis_name="c", subcore_axis_name="s",
                                    num_cores=2, num_subcores=16)
    f32 = jnp.float32
    kern = functools.partial(
        pl.kernel, mesh=scmesh,
        compiler_params=pltpu.CompilerParams(use_tc_tiling_on_sc=False,
                                             needs_layout_passes=False),
        out_type=[jax.ShapeDtypeStruct((2, 4 * OUTP), f32),
                  jax.ShapeDtypeStruct((2, OUTP), f32),
                  jax.ShapeDtypeStruct((2, 4 * OUTP), f32),
                  jax.ShapeDtypeStruct((2, OUTP), f32)],
        scratch_types=[
            pltpu.VMEM((NUM,), f32),
            pltpu.VMEM((PRE,), f32),
            pltpu.VMEM((PRE,), f32),
            pltpu.VMEM((PRE,), jnp.int32),
            pltpu.VMEM((PRE,), jnp.int32),
            pltpu.VMEM((PRE,), jnp.int32),
            pltpu.VMEM((16 * 256,), jnp.int32),
            pltpu.VMEM((256,), jnp.int32),
            pltpu.VMEM((256,), jnp.int32),
            pltpu.VMEM((PRE,), f32),
            pltpu.VMEM((PRE,), f32),
            pltpu.VMEM((PRE,), f32),
            pltpu.VMEM((PRE,), f32),
            pltpu.VMEM((PRE,), jnp.int32),
            pltpu.VMEM((PRE,), f32),
            pltpu.VMEM((PRE,), f32),
            pltpu.VMEM((PRE,), f32),
            pltpu.VMEM((PRE,), f32),
            pltpu.VMEM((PRE,), f32),
            pltpu.VMEM((PRE,), f32),
            pltpu.VMEM((PRE,), jnp.int32),
            pltpu.VMEM((OUTP,), jnp.int32),
            pltpu.VMEM((4 * OUTP,), f32),
            pltpu.VMEM((OUTP,), f32),
            pltpu.VMEM_SHARED((OUTP,), jnp.int32),
        ])(_kernel_body)
    return kern(obj_l, obj_r, anch_l, anch_r, reg_l, reg_r)


def kernel(anchors_left, anchors_right, objectness_left, objectness_right,
           box_regression_left, box_regression_right):
    N = objectness_left.shape[0]
    obj_l = jnp.transpose(objectness_left, (0, 2, 3, 1)).reshape(-1)
    obj_r = jnp.transpose(objectness_right, (0, 2, 3, 1)).reshape(-1)
    reg_l = jnp.transpose(box_regression_left.reshape(N, 3, 4, 128, 128),
                          (0, 3, 4, 1, 2)).reshape(-1)
    reg_r = jnp.transpose(box_regression_right.reshape(N, 3, 4, 128, 128),
                          (0, 3, 4, 1, 2)).reshape(-1)
    anch_l = anchors_left.reshape(-1)
    anch_r = anchors_right.reshape(-1)
    obl, osl, obr, osr = _run(obj_l, obj_r, anch_l, anch_r, reg_l, reg_r)
    bl = jnp.transpose(obl.reshape(N, 4, OUTP), (0, 2, 1))[:, :POST, :]
    sl = osl[:, :POST]
    br = jnp.transpose(obr.reshape(N, 4, OUTP), (0, 2, 1))[:, :POST, :]
    sr = osr[:, :POST]
    return (bl, sl, br, sr)
